# baseline (speedup 1.0000x reference)
import functools

import jax
import jax.numpy as jnp
from jax import lax
from jax.experimental import pallas as pl
from jax.experimental.pallas import tpu as pltpu
from jax.experimental.pallas import tpu_sc as plsc

C = 1000
L = 50
B = 1024
RPT = 40
TPL = C // RPT
NTASK = TPL * L
NC = 2
NS = 16
NW = NC * NS
NPAIR = ((NTASK + NW - 1) // NW + 1) // 2

_mesh = plsc.VectorSubcoreMesh(core_axis_name="c", subcore_axis_name="s")


def _sc_body(xt_hbm, out_hbm, xcol0, xcol1, buf0, buf1, sem0, sem1):
    wid = lax.axis_index("s") * NC + lax.axis_index("c")
    iota = lax.broadcasted_iota(jnp.int32, (16,), 0)
    ones = jnp.full((16,), 1.0, jnp.float32)
    zeros = jnp.zeros((16,), jnp.float32)

    s = (wid * NTASK) // NW
    e = ((wid + 1) * NTASK) // NW

    def _zero(k, c):
        r = k // (B // 16)
        o = (k % (B // 16)) * 16
        buf0[r, pl.ds(o, 16)] = zeros
        buf1[r, pl.ds(o, 16)] = zeros
        return c

    lax.fori_loop(0, RPT * (B // 16), _zero, 0)

    def _scatter(buf, xcol, t, val):
        c0 = (t % TPL) * RPT

        def _chunk(k, c):
            xv = xcol[k // 8, pl.ds((k % 8) * 16, 16)]
            cl = xv - c0
            m = (cl >= 0) & (cl < RPT)
            cl = jnp.clip(cl, 0, RPT - 1)
            bidx = (k // 8) * 128 + (k % 8) * 16 + iota
            plsc.store_scatter(buf, [cl, bidx], val, mask=m)
            return c

        lax.fori_loop(0, B // 16, _chunk, 0)

    def _start(buf, t, sem):
        l = t // TPL
        c0 = pl.multiple_of((t % TPL) * RPT, RPT)
        pltpu.make_async_copy(buf, out_hbm.at[l, pl.ds(c0, RPT)], sem).start()

    def _wait(buf, sem):
        pltpu.make_async_copy(buf, out_hbm.at[0, pl.ds(0, RPT)], sem).wait()

    def _task(buf, xcol, sem, i, j):
        t = s + j
        tp = t - 2

        @pl.when(jnp.logical_and(i > 0, t < e))
        def _():
            _wait(buf, sem)
            _scatter(buf, xcol, tp, zeros)

        @pl.when(t < e)
        def _():
            @pl.when(jnp.logical_or(i == 0, t // TPL != tp // TPL))
            def _():
                pltpu.sync_copy(xt_hbm.at[t // TPL], xcol)

            _scatter(buf, xcol, t, ones)
            _start(buf, t, sem)

    def _pair(i, c):
        _task(buf0, xcol0, sem0, i, 2 * i)
        _task(buf1, xcol1, sem1, i, 2 * i + 1)
        return c

    lax.fori_loop(0, NPAIR, _pair, 0)

    _wait(buf0, sem0)
    _wait(buf1, sem1)


def kernel(x):
    xt = jnp.swapaxes(x, 0, 1).reshape(L, 8, 128)
    f = functools.partial(
        pl.kernel,
        mesh=_mesh,
        compiler_params=pltpu.CompilerParams(needs_layout_passes=False),
        out_type=jax.ShapeDtypeStruct((L, C, B), jnp.float32),
        scratch_types=[
            pltpu.VMEM((8, 128), jnp.int32),
            pltpu.VMEM((8, 128), jnp.int32),
            pltpu.VMEM((RPT, B), jnp.float32),
            pltpu.VMEM((RPT, B), jnp.float32),
            pltpu.SemaphoreType.DMA,
            pltpu.SemaphoreType.DMA,
        ],
    )(_sc_body)
    p = f(xt)
    return jnp.transpose(p, (2, 1, 0))


# speedup vs baseline: 1.4827x; 1.4827x over previous
import jax
import jax.numpy as jnp
from jax.experimental import pallas as pl

NUM_CLASSES = 1000


def _body(xt_ref, o_ref):
    cls = jax.lax.broadcasted_iota(jnp.int32, o_ref.shape, 1)
    o_ref[...] = (cls == xt_ref[...]).astype(jnp.float32)


def kernel(x):
    B, L = x.shape
    xt = jnp.swapaxes(x, 0, 1).reshape(L, 1, B)
    p = pl.pallas_call(
        _body,
        grid=(L,),
        in_specs=[pl.BlockSpec((1, 1, B), lambda i: (i, 0, 0))],
        out_specs=pl.BlockSpec((1, NUM_CLASSES, B), lambda i: (i, 0, 0)),
        out_shape=jax.ShapeDtypeStruct((L, NUM_CLASSES, B), jnp.float32),
    )(xt)
    return jnp.transpose(p, (2, 1, 0))
